# baseline (speedup 1.0000x reference)
import jax
import jax.numpy as jnp
from jax.experimental import pallas as pl
from jax.experimental.pallas import tpu as pltpu

_SEQ_TILE = 2048


def _ln_kernel(x_ref, pos_ref, gamma_ref, beta_ref, out_ref):
    e = x_ref[0] + pos_ref[...]
    d_inv = 1.0 / e.shape[1]
    mean = jnp.sum(e, axis=1, keepdims=True) * d_inv
    meansq = jnp.sum(e * e, axis=1, keepdims=True) * d_inv
    var = meansq - mean * mean
    inv = jax.lax.rsqrt(var + 1e-12)
    out_ref[0] = (e - mean) * (inv * gamma_ref[...]) + beta_ref[...]


def kernel(x, pos_table, gamma, beta):
    B, S, D = x.shape
    ts = _SEQ_TILE
    gamma2 = gamma.reshape(1, D)
    beta2 = beta.reshape(1, D)
    grid = (S // ts, B)
    return pl.pallas_call(
        _ln_kernel,
        grid=grid,
        in_specs=[
            pl.BlockSpec((1, ts, D), lambda s, b: (b, s, 0)),
            pl.BlockSpec((ts, D), lambda s, b: (s, 0)),
            pl.BlockSpec((1, D), lambda s, b: (0, 0)),
            pl.BlockSpec((1, D), lambda s, b: (0, 0)),
        ],
        out_specs=pl.BlockSpec((1, ts, D), lambda s, b: (b, s, 0)),
        out_shape=jax.ShapeDtypeStruct((B, S, D), x.dtype),
        compiler_params=pltpu.CompilerParams(
            dimension_semantics=("parallel", "parallel"),
        ),
    )(x, pos_table, gamma2, beta2)


# speedup vs baseline: 1.0479x; 1.0479x over previous
import jax
import jax.numpy as jnp
from jax.experimental import pallas as pl
from jax.experimental.pallas import tpu as pltpu

_SEQ_TILE = 2048


def _ln_kernel(x_ref, pos_ref, gamma_ref, beta_ref, out_ref):
    s = pl.program_id(0)
    ts = x_ref.shape[1]
    e = x_ref[0] + pos_ref[pl.ds(s * ts, ts), :]
    d_inv = 1.0 / e.shape[1]
    mean = jnp.sum(e, axis=1, keepdims=True) * d_inv
    meansq = jnp.sum(e * e, axis=1, keepdims=True) * d_inv
    var = meansq - mean * mean
    inv = jax.lax.rsqrt(var + 1e-12)
    out_ref[0] = (e - mean) * (inv * gamma_ref[...]) + beta_ref[...]


def kernel(x, pos_table, gamma, beta):
    B, S, D = x.shape
    ts = _SEQ_TILE
    gamma2 = gamma.reshape(1, D)
    beta2 = beta.reshape(1, D)
    grid = (S // ts, B)
    return pl.pallas_call(
        _ln_kernel,
        grid=grid,
        in_specs=[
            pl.BlockSpec((1, ts, D), lambda s, b: (b, s, 0)),
            pl.BlockSpec((S, D), lambda s, b: (0, 0)),
            pl.BlockSpec((1, D), lambda s, b: (0, 0)),
            pl.BlockSpec((1, D), lambda s, b: (0, 0)),
        ],
        out_specs=pl.BlockSpec((1, ts, D), lambda s, b: (b, s, 0)),
        out_shape=jax.ShapeDtypeStruct((B, S, D), x.dtype),
        compiler_params=pltpu.CompilerParams(
            dimension_semantics=("parallel", "parallel"),
        ),
    )(x, pos_table, gamma2, beta2)
